# Initial kernel scaffold; baseline (speedup 1.0000x reference)
import functools

import jax
import jax.numpy as jnp
from jax import lax
from jax.experimental import pallas as pl
from jax.experimental.pallas import tpu as pltpu
from jax.experimental.pallas import tpu_sc as plsc

N = 10000
E = 320000
D = 128
NC = 2
NS = 16
NW = NC * NS
E_PER_W = E // NW
CH = 80
NCH = E_PER_W // CH
RPS = N // NS
ZR = 25
DEGW = 16


def _sc_agg_body(with_deg, *refs):
    if with_deg:
        (h_hbm, src_hbm, dst_hbm, agg_out, deg_out,
         src_v, dst_v, rows_v, ones_v, zrow_v, zdeg_v, accum_sh, deg_sh) = refs
    else:
        (h_hbm, src_hbm, dst_hbm, agg_out,
         src_v, dst_v, rows_v, zrow_v, accum_sh) = refs

    cid = lax.axis_index("c")
    sid = lax.axis_index("s")
    wid = sid * NC + cid

    @pl.loop(0, ZR)
    def _(r):
        @pl.loop(0, D // 16)
        def _(c):
            zrow_v[r, pl.ds(c * 16, 16)] = jnp.zeros((16,), jnp.float32)

    if with_deg:
        @pl.loop(0, NCH)
        def _(r):
            ones_v[r, :] = jnp.ones((DEGW,), jnp.float32)

        @pl.loop(0, NCH)
        def _(r):
            zdeg_v[r, :] = jnp.zeros((DEGW,), jnp.float32)

    @pl.loop(0, RPS // ZR)
    def _(j):
        pltpu.sync_copy(zrow_v, accum_sh.at[pl.ds(sid * RPS + j * ZR, ZR), :])

    if with_deg:
        @pl.loop(0, RPS // NCH)
        def _(j):
            pltpu.sync_copy(zdeg_v, deg_sh.at[pl.ds(sid * RPS + j * NCH, NCH), :])

    plsc.subcore_barrier()

    pltpu.sync_copy(src_hbm.at[wid], src_v)
    pltpu.sync_copy(dst_hbm.at[wid], dst_v)

    @pl.loop(0, NCH)
    def _(g):
        pltpu.sync_copy(h_hbm.at[src_v.at[g]], rows_v)
        pltpu.sync_copy(rows_v, accum_sh.at[dst_v.at[g]], add=True)
        if with_deg:
            pltpu.sync_copy(ones_v.at[pl.ds(0, CH)], deg_sh.at[dst_v.at[g]],
                            add=True)

    plsc.subcore_barrier()

    rs = pl.ds(sid * RPS, RPS)
    pltpu.sync_copy(accum_sh.at[rs, :], agg_out.at[cid, rs, :])
    if with_deg:
        pltpu.sync_copy(deg_sh.at[rs, :], deg_out.at[cid, rs, :])


def _make_sc_agg(with_deg):
    mesh = plsc.VectorSubcoreMesh(core_axis_name="c", subcore_axis_name="s",
                                  num_cores=NC, num_subcores=NS)
    out_type = [jax.ShapeDtypeStruct((NC, N, D), jnp.float32)]
    scratch = [
        pltpu.VMEM((NCH, CH), jnp.int32),
        pltpu.VMEM((NCH, CH), jnp.int32),
        pltpu.VMEM((CH, D), jnp.float32),
    ]
    if with_deg:
        out_type.append(jax.ShapeDtypeStruct((NC, N, DEGW), jnp.float32))
        scratch.append(pltpu.VMEM((NCH, DEGW), jnp.float32))
    scratch.append(pltpu.VMEM((ZR, D), jnp.float32))
    if with_deg:
        scratch.append(pltpu.VMEM((NCH, DEGW), jnp.float32))
    scratch.append(pltpu.VMEM_SHARED((N, D), jnp.float32))
    if with_deg:
        scratch.append(pltpu.VMEM_SHARED((N, DEGW), jnp.float32))
    return pl.kernel(functools.partial(_sc_agg_body, with_deg),
                     out_type=tuple(out_type) if with_deg else out_type[0],
                     mesh=mesh, scratch_types=scratch)


def _dense_body(relu, h_ref, aggp_ref, degp_ref, ws_ref, wn_ref, b_ref, o_ref):
    p = aggp_ref[0] + aggp_ref[1]
    deg = degp_ref[0] + degp_ref[1]
    hn = p * (1.0 / jnp.maximum(deg[:, 0:1], 1.0))
    acc = jnp.dot(h_ref[...], ws_ref[...], preferred_element_type=jnp.float32)
    acc = acc + jnp.dot(hn, wn_ref[...], preferred_element_type=jnp.float32)
    acc = acc + b_ref[...]
    if relu:
        acc = jnp.maximum(acc, 0.0)
    o_ref[...] = acc


BLK = 1000


def _dense(h, aggp, degp, w_self, w_neigh, b, relu):
    grid = (N // BLK,)
    return pl.pallas_call(
        functools.partial(_dense_body, relu),
        grid=grid,
        in_specs=[
            pl.BlockSpec((BLK, D), lambda i: (i, 0)),
            pl.BlockSpec((NC, BLK, D), lambda i: (0, i, 0)),
            pl.BlockSpec((NC, BLK, DEGW), lambda i: (0, i, 0)),
            pl.BlockSpec((D, D), lambda i: (0, 0)),
            pl.BlockSpec((D, D), lambda i: (0, 0)),
            pl.BlockSpec((1, D), lambda i: (0, 0)),
        ],
        out_specs=pl.BlockSpec((BLK, D), lambda i: (i, 0)),
        out_shape=jax.ShapeDtypeStruct((N, D), jnp.float32),
    )(h, aggp, degp, w_self, w_neigh, b)


def kernel(x, edge_index, W1_self, W1_neigh, b1, W2_self, W2_neigh, b2):
    src = edge_index[0].reshape(NW, NCH, CH)
    dst = edge_index[1].reshape(NW, NCH, CH)
    b1r = b1.reshape(1, D)
    b2r = b2.reshape(1, D)

    agg1p, degp = _make_sc_agg(True)(x, src, dst)
    h1 = _dense(x, agg1p, degp, W1_self, W1_neigh, b1r, relu=True)
    agg2p = _make_sc_agg(False)(h1, src, dst)
    out = _dense(h1, agg2p, degp, W2_self, W2_neigh, b2r, relu=False)
    return out


# speedup vs baseline: 7.5641x; 7.5641x over previous
import functools

import jax
import jax.numpy as jnp
from jax import lax
from jax.experimental import pallas as pl
from jax.experimental.pallas import tpu as pltpu
from jax.experimental.pallas import tpu_sc as plsc

N = 10000
NP = 10240
E = 320000
D = 128
NC = 2
NS = 16
NW = NC * NS
E_PER_W = E // NW
CH = 100
NCH = E_PER_W // CH
RPS = NP // NS
ZR = 32

_MESH = dict(core_axis_name="c", subcore_axis_name="s",
             num_cores=NC, num_subcores=NS)


def _fill(ref, rows, value):
    @pl.loop(0, rows)
    def _(r):
        @pl.loop(0, D // 16)
        def _(c):
            ref[r, pl.ds(c * 16, 16)] = jnp.full((16,), value, jnp.float32)


def _sc_pass_body(gather, *refs):
    if gather:
        (h_hbm, src_hbm, dst_hbm, out_hbm,
         src_v, dst_v, rows_v, zrow_v, accum_sh) = refs
    else:
        (dst_hbm, out_hbm, dst_v, rows_v, zrow_v, accum_sh) = refs

    cid = lax.axis_index("c")
    sid = lax.axis_index("s")
    wid = sid * NC + cid

    _fill(zrow_v, ZR, 0.0)
    if not gather:
        _fill(rows_v, CH, 1.0)

    @pl.loop(0, RPS // ZR)
    def _(j):
        pltpu.sync_copy(zrow_v, accum_sh.at[pl.ds(sid * RPS + j * ZR, ZR), :])

    plsc.subcore_barrier()

    if gather:
        pltpu.sync_copy(src_hbm.at[wid], src_v)
    pltpu.sync_copy(dst_hbm.at[wid], dst_v)

    @pl.loop(0, NCH)
    def _(g):
        if gather:
            pltpu.sync_copy(h_hbm.at[src_v.at[g]], rows_v)
        pltpu.sync_copy(rows_v, accum_sh.at[dst_v.at[g]], add=True)

    plsc.subcore_barrier()

    rs = pl.ds(sid * RPS, RPS)
    pltpu.sync_copy(accum_sh.at[rs, :], out_hbm.at[cid, rs, :])


def _make_sc_pass(gather):
    scratch = [
        pltpu.VMEM((NCH, CH), jnp.int32),
        pltpu.VMEM((NCH, CH), jnp.int32),
        pltpu.VMEM((CH, D), jnp.float32),
        pltpu.VMEM((ZR, D), jnp.float32),
        pltpu.VMEM_SHARED((NP, D), jnp.float32),
    ]
    if not gather:
        scratch = scratch[1:]
    return pl.kernel(
        functools.partial(_sc_pass_body, gather),
        out_type=jax.ShapeDtypeStruct((NC, NP, D), jnp.float32),
        mesh=plsc.VectorSubcoreMesh(**_MESH),
        scratch_types=scratch)


def _dense_body(relu, h_ref, aggp_ref, degp_ref, ws_ref, wn_ref, b_ref, o_ref):
    p = aggp_ref[0] + aggp_ref[1]
    deg = degp_ref[0, :, 0:1] + degp_ref[1, :, 0:1]
    hn = p * (1.0 / jnp.maximum(deg, 1.0))
    acc = jnp.dot(h_ref[...], ws_ref[...], preferred_element_type=jnp.float32)
    acc = acc + jnp.dot(hn, wn_ref[...], preferred_element_type=jnp.float32)
    acc = acc + b_ref[...]
    if relu:
        acc = jnp.maximum(acc, 0.0)
    o_ref[...] = acc


BLK = 1000


def _dense(h, aggp, degp, w_self, w_neigh, b, relu):
    return pl.pallas_call(
        functools.partial(_dense_body, relu),
        grid=(N // BLK,),
        in_specs=[
            pl.BlockSpec((BLK, D), lambda i: (i, 0)),
            pl.BlockSpec((NC, BLK, D), lambda i: (0, i, 0)),
            pl.BlockSpec((NC, BLK, D), lambda i: (0, i, 0)),
            pl.BlockSpec((D, D), lambda i: (0, 0)),
            pl.BlockSpec((D, D), lambda i: (0, 0)),
            pl.BlockSpec((1, D), lambda i: (0, 0)),
        ],
        out_specs=pl.BlockSpec((BLK, D), lambda i: (i, 0)),
        out_shape=jax.ShapeDtypeStruct((N, D), jnp.float32),
    )(h, aggp, degp, w_self, w_neigh, b)


def kernel(x, edge_index, W1_self, W1_neigh, b1, W2_self, W2_neigh, b2):
    src = edge_index[0].reshape(NW, NCH, CH)
    dst = edge_index[1].reshape(NW, NCH, CH)
    b1r = b1.reshape(1, D)
    b2r = b2.reshape(1, D)

    degp = _make_sc_pass(False)(dst)
    agg1p = _make_sc_pass(True)(x, src, dst)
    h1 = _dense(x, agg1p, degp, W1_self, W1_neigh, b1r, relu=True)
    agg2p = _make_sc_pass(True)(h1, src, dst)
    out = _dense(h1, agg2p, degp, W2_self, W2_neigh, b2r, relu=False)
    return out


# speedup vs baseline: 10.2214x; 1.3513x over previous
import functools

import jax
import jax.numpy as jnp
from jax import lax
from jax.experimental import pallas as pl
from jax.experimental.pallas import tpu as pltpu
from jax.experimental.pallas import tpu_sc as plsc

N = 10000
NP = 10240
E = 320000
D = 128
NC = 2
NS = 16
NW = NC * NS
E_PER_W = E // NW
CH = 80
NCH = E_PER_W // CH
RPS = NP // NS
ZR = 8

_MESH = dict(core_axis_name="c", subcore_axis_name="s",
             num_cores=NC, num_subcores=NS)


def _fill(ref, rows, value):
    @pl.loop(0, rows)
    def _(r):
        @pl.loop(0, D // 16)
        def _(c):
            ref[r, pl.ds(c * 16, 16)] = jnp.full((16,), value, jnp.float32)


def _zero_accum(sid, zrow_v, accum_sh):
    _fill(zrow_v, ZR, 0.0)

    @pl.loop(0, RPS // ZR)
    def _(j):
        pltpu.sync_copy(zrow_v, accum_sh.at[pl.ds(sid * RPS + j * ZR, ZR), :])


def _sc_agg_body(h_hbm, srcf_hbm, dst_hbm, out_hbm,
                 sidx_v, dst_v, rows0_v, rows1_v, zrow_v, accum_sh,
                 isem0, isem1, isem2, isem3, rsem0, rsem1):
    cid = lax.axis_index("c")
    sid = lax.axis_index("s")
    wid = sid * NC + cid
    ebase = wid * E_PER_W

    isems = (isem0, isem1, isem2, isem3)
    rows = (rows0_v, rows1_v)
    rsems = (rsem0, rsem1)

    _zero_accum(sid, zrow_v, accum_sh)
    plsc.subcore_barrier()

    pltpu.sync_copy(dst_hbm.at[wid], dst_v)

    def idx_load(m, s):
        off = pl.multiple_of(ebase + m * CH, 8)
        pltpu.async_copy(srcf_hbm.at[pl.ds(off, CH)], sidx_v.at[s], isems[s])

    def idx_wait(s):
        pltpu.make_async_copy(srcf_hbm.at[pl.ds(0, CH)], sidx_v.at[s],
                              isems[s]).wait()

    def gather_start(s, r):
        pltpu.async_copy(h_hbm.at[sidx_v.at[s]], rows[r], rsems[r])

    def gather_wait(r):
        pltpu.make_async_copy(h_hbm.at[sidx_v.at[0]], rows[r], rsems[r]).wait()

    for s in range(4):
        idx_load(s, s)
    idx_wait(0)
    gather_start(0, 0)
    idx_wait(1)
    gather_start(1, 1)

    @pl.loop(0, NCH // 4)
    def _(q):
        for b in range(4):
            m = q * 4 + b
            r = b % 2
            gather_wait(r)
            pltpu.sync_copy(rows[r], accum_sh.at[dst_v.at[m]], add=True)

            @pl.when(m + 4 < NCH)
            def _():
                idx_load(m + 4, b)

            @pl.when(m + 2 < NCH)
            def _():
                idx_wait((b + 2) % 4)
                gather_start((b + 2) % 4, r)

    for b in range(NCH % 4):
        m = (NCH // 4) * 4 + b
        gather_wait(b % 2)
        pltpu.sync_copy(rows[b % 2], accum_sh.at[dst_v.at[m]], add=True)

    plsc.subcore_barrier()

    rs = pl.ds(sid * RPS, RPS)
    pltpu.sync_copy(accum_sh.at[rs, :], out_hbm.at[cid, rs, :])


def _sc_deg_body(dst_hbm, out_hbm, dst_v, ones_v, zrow_v, accum_sh):
    cid = lax.axis_index("c")
    sid = lax.axis_index("s")
    wid = sid * NC + cid

    _zero_accum(sid, zrow_v, accum_sh)
    _fill(ones_v, CH, 1.0)
    plsc.subcore_barrier()

    pltpu.sync_copy(dst_hbm.at[wid], dst_v)

    @pl.loop(0, NCH)
    def _(g):
        pltpu.sync_copy(ones_v, accum_sh.at[dst_v.at[g]], add=True)

    plsc.subcore_barrier()

    rs = pl.ds(sid * RPS, RPS)
    pltpu.sync_copy(accum_sh.at[rs, :], out_hbm.at[cid, rs, :])


def _make_sc_agg():
    return pl.kernel(
        _sc_agg_body,
        out_type=jax.ShapeDtypeStruct((NC, NP, D), jnp.float32),
        mesh=plsc.VectorSubcoreMesh(**_MESH),
        scratch_types=[
            pltpu.VMEM((4, CH), jnp.int32),
            pltpu.VMEM((NCH, CH), jnp.int32),
            pltpu.VMEM((CH, D), jnp.float32),
            pltpu.VMEM((CH, D), jnp.float32),
            pltpu.VMEM((ZR, D), jnp.float32),
            pltpu.VMEM_SHARED((NP, D), jnp.float32),
            pltpu.SemaphoreType.DMA,
            pltpu.SemaphoreType.DMA,
            pltpu.SemaphoreType.DMA,
            pltpu.SemaphoreType.DMA,
            pltpu.SemaphoreType.DMA,
            pltpu.SemaphoreType.DMA,
        ])


def _make_sc_deg():
    return pl.kernel(
        _sc_deg_body,
        out_type=jax.ShapeDtypeStruct((NC, NP, D), jnp.float32),
        mesh=plsc.VectorSubcoreMesh(**_MESH),
        scratch_types=[
            pltpu.VMEM((NCH, CH), jnp.int32),
            pltpu.VMEM((CH, D), jnp.float32),
            pltpu.VMEM((ZR, D), jnp.float32),
            pltpu.VMEM_SHARED((NP, D), jnp.float32),
        ])


def _dense_body(relu, h_ref, aggp_ref, degp_ref, ws_ref, wn_ref, b_ref, o_ref):
    p = aggp_ref[0] + aggp_ref[1]
    deg = degp_ref[0, :, 0:1] + degp_ref[1, :, 0:1]
    hn = p * (1.0 / jnp.maximum(deg, 1.0))
    acc = jnp.dot(h_ref[...], ws_ref[...], preferred_element_type=jnp.float32)
    acc = acc + jnp.dot(hn, wn_ref[...], preferred_element_type=jnp.float32)
    acc = acc + b_ref[...]
    if relu:
        acc = jnp.maximum(acc, 0.0)
    o_ref[...] = acc


BLK = 1000


def _dense(h, aggp, degp, w_self, w_neigh, b, relu):
    return pl.pallas_call(
        functools.partial(_dense_body, relu),
        grid=(N // BLK,),
        in_specs=[
            pl.BlockSpec((BLK, D), lambda i: (i, 0)),
            pl.BlockSpec((NC, BLK, D), lambda i: (0, i, 0)),
            pl.BlockSpec((NC, BLK, D), lambda i: (0, i, 0)),
            pl.BlockSpec((D, D), lambda i: (0, 0)),
            pl.BlockSpec((D, D), lambda i: (0, 0)),
            pl.BlockSpec((1, D), lambda i: (0, 0)),
        ],
        out_specs=pl.BlockSpec((BLK, D), lambda i: (i, 0)),
        out_shape=jax.ShapeDtypeStruct((N, D), jnp.float32),
    )(h, aggp, degp, w_self, w_neigh, b)


def kernel(x, edge_index, W1_self, W1_neigh, b1, W2_self, W2_neigh, b2):
    srcf = edge_index[0]
    dst = edge_index[1].reshape(NW, NCH, CH)
    b1r = b1.reshape(1, D)
    b2r = b2.reshape(1, D)

    degp = _make_sc_deg()(dst)
    agg1p = _make_sc_agg()(x, srcf, dst)
    h1 = _dense(x, agg1p, degp, W1_self, W1_neigh, b1r, relu=True)
    agg2p = _make_sc_agg()(h1, srcf, dst)
    out = _dense(h1, agg2p, degp, W2_self, W2_neigh, b2r, relu=False)
    return out


# speedup vs baseline: 10.4548x; 1.0228x over previous
import functools

import jax
import jax.numpy as jnp
from jax import lax
from jax.experimental import pallas as pl
from jax.experimental.pallas import tpu as pltpu
from jax.experimental.pallas import tpu_sc as plsc

N = 10000
NP = 10240
E = 320000
D = 128
NC = 2
NS = 16
NW = NC * NS
E_PER_W = E // NW
CH = 80
NCH = E_PER_W // CH
RPS = NP // NS
ZR = 8

_MESH = dict(core_axis_name="c", subcore_axis_name="s",
             num_cores=NC, num_subcores=NS)


def _fill(ref, rows, value):
    @pl.loop(0, rows)
    def _(r):
        @pl.loop(0, D // 16)
        def _(c):
            ref[r, pl.ds(c * 16, 16)] = jnp.full((16,), value, jnp.float32)


def _zero_accum(sid, zrow_v, accum_sh):
    _fill(zrow_v, ZR, 0.0)

    @pl.loop(0, RPS // ZR)
    def _(j):
        pltpu.sync_copy(zrow_v, accum_sh.at[pl.ds(sid * RPS + j * ZR, ZR), :])


def _sc_agg_body(h_hbm, srcf_hbm, dstf_hbm, out_hbm,
                 sidx_v, didx_v, rows0_v, rows1_v, rows2_v, rows3_v,
                 zrow_v, accum_sh, *sems):
    cid = lax.axis_index("c")
    sid = lax.axis_index("s")
    wid = sid * NC + cid
    ebase = wid * E_PER_W

    isems = sems[0:8]
    rsems = sems[8:12]
    ssems = sems[12:16]
    rows = (rows0_v, rows1_v, rows2_v, rows3_v)

    _zero_accum(sid, zrow_v, accum_sh)
    plsc.subcore_barrier()

    def idx_load(m, s):
        off = pl.multiple_of(ebase + m * CH, 8)
        pltpu.async_copy(srcf_hbm.at[pl.ds(off, CH)], sidx_v.at[s], isems[s])
        pltpu.async_copy(dstf_hbm.at[pl.ds(off, CH)], didx_v.at[s], isems[s])

    def idx_wait(s):
        pltpu.make_async_copy(srcf_hbm.at[pl.ds(0, CH)], sidx_v.at[s],
                              isems[s]).wait()
        pltpu.make_async_copy(dstf_hbm.at[pl.ds(0, CH)], didx_v.at[s],
                              isems[s]).wait()

    def gather_start(s, r):
        pltpu.async_copy(h_hbm.at[sidx_v.at[s]], rows[r], rsems[r])

    def gather_wait(r):
        pltpu.make_async_copy(h_hbm.at[sidx_v.at[0]], rows[r], rsems[r]).wait()

    def scatter_start(s, r):
        pltpu.async_copy(rows[r], accum_sh.at[didx_v.at[s]], ssems[r],
                         add=True)

    def scatter_wait(r):
        pltpu.make_async_copy(rows[r], accum_sh.at[didx_v.at[0]],
                              ssems[r]).wait()

    def body(m, guard):
        b8, b4 = m % 8, m % 4
        gather_wait(b4)
        scatter_start(b8, b4)
        if m >= 2:
            scatter_wait((m + 2) % 4)
            if guard and m + 6 < NCH:
                idx_load(m + 6, (m + 6) % 8)
        if guard and m + 2 < NCH:
            idx_wait((m + 2) % 8)
            gather_start((m + 2) % 8, (m + 2) % 4)

    for s in range(8):
        idx_load(s, s)
    idx_wait(0)
    gather_start(0, 0)
    idx_wait(1)
    gather_start(1, 1)

    for m in range(8):
        body(m, True)

    @pl.loop(1, NCH // 8)
    def _(q):
        for b in range(8):
            m = q * 8 + b

            def dyn_body(b=b, m=m):
                b4 = b % 4
                gather_wait(b4)
                scatter_start(b, b4)
                scatter_wait((b + 2) % 4)

                @pl.when(m + 6 < NCH)
                def _():
                    idx_load(m + 6, (b + 6) % 8)

                @pl.when(m + 2 < NCH)
                def _():
                    idx_wait((b + 2) % 8)
                    gather_start((b + 2) % 8, (b + 2) % 4)

            dyn_body()

    for m in range((NCH // 8) * 8, NCH):
        body(m, True)

    scatter_wait((NCH - 2) % 4)
    scatter_wait((NCH - 1) % 4)

    plsc.subcore_barrier()

    rs = pl.ds(sid * RPS, RPS)
    pltpu.sync_copy(accum_sh.at[rs, :], out_hbm.at[cid, rs, :])


def _sc_deg_body(dst_hbm, out_hbm, dst_v, ones_v, zrow_v, accum_sh):
    cid = lax.axis_index("c")
    sid = lax.axis_index("s")
    wid = sid * NC + cid

    _zero_accum(sid, zrow_v, accum_sh)
    _fill(ones_v, CH, 1.0)
    plsc.subcore_barrier()

    pltpu.sync_copy(dst_hbm.at[wid], dst_v)

    @pl.loop(0, NCH)
    def _(g):
        pltpu.sync_copy(ones_v, accum_sh.at[dst_v.at[g]], add=True)

    plsc.subcore_barrier()

    rs = pl.ds(sid * RPS, RPS)
    pltpu.sync_copy(accum_sh.at[rs, :], out_hbm.at[cid, rs, :])


def _make_sc_agg():
    return pl.kernel(
        _sc_agg_body,
        out_type=jax.ShapeDtypeStruct((NC, NP, D), jnp.float32),
        mesh=plsc.VectorSubcoreMesh(**_MESH),
        scratch_types=[
            pltpu.VMEM((8, CH), jnp.int32),
            pltpu.VMEM((8, CH), jnp.int32),
            pltpu.VMEM((CH, D), jnp.float32),
            pltpu.VMEM((CH, D), jnp.float32),
            pltpu.VMEM((CH, D), jnp.float32),
            pltpu.VMEM((CH, D), jnp.float32),
            pltpu.VMEM((ZR, D), jnp.float32),
            pltpu.VMEM_SHARED((NP, D), jnp.float32),
        ] + [pltpu.SemaphoreType.DMA] * 16)


def _make_sc_deg():
    return pl.kernel(
        _sc_deg_body,
        out_type=jax.ShapeDtypeStruct((NC, NP, D), jnp.float32),
        mesh=plsc.VectorSubcoreMesh(**_MESH),
        scratch_types=[
            pltpu.VMEM((NCH, CH), jnp.int32),
            pltpu.VMEM((CH, D), jnp.float32),
            pltpu.VMEM((ZR, D), jnp.float32),
            pltpu.VMEM_SHARED((NP, D), jnp.float32),
        ])


def _dense_body(relu, h_ref, aggp_ref, degp_ref, ws_ref, wn_ref, b_ref, o_ref):
    p = aggp_ref[0] + aggp_ref[1]
    deg = degp_ref[0, :, 0:1] + degp_ref[1, :, 0:1]
    hn = p * (1.0 / jnp.maximum(deg, 1.0))
    acc = jnp.dot(h_ref[...], ws_ref[...], preferred_element_type=jnp.float32)
    acc = acc + jnp.dot(hn, wn_ref[...], preferred_element_type=jnp.float32)
    acc = acc + b_ref[...]
    if relu:
        acc = jnp.maximum(acc, 0.0)
    o_ref[...] = acc


BLK = 1000


def _dense(h, aggp, degp, w_self, w_neigh, b, relu):
    return pl.pallas_call(
        functools.partial(_dense_body, relu),
        grid=(N // BLK,),
        in_specs=[
            pl.BlockSpec((BLK, D), lambda i: (i, 0)),
            pl.BlockSpec((NC, BLK, D), lambda i: (0, i, 0)),
            pl.BlockSpec((NC, BLK, D), lambda i: (0, i, 0)),
            pl.BlockSpec((D, D), lambda i: (0, 0)),
            pl.BlockSpec((D, D), lambda i: (0, 0)),
            pl.BlockSpec((1, D), lambda i: (0, 0)),
        ],
        out_specs=pl.BlockSpec((BLK, D), lambda i: (i, 0)),
        out_shape=jax.ShapeDtypeStruct((N, D), jnp.float32),
    )(h, aggp, degp, w_self, w_neigh, b)


def kernel(x, edge_index, W1_self, W1_neigh, b1, W2_self, W2_neigh, b2):
    srcf = edge_index[0]
    dstf = edge_index[1]
    dst3 = dstf.reshape(NW, NCH, CH)
    b1r = b1.reshape(1, D)
    b2r = b2.reshape(1, D)

    degp = _make_sc_deg()(dst3)
    agg1p = _make_sc_agg()(x, srcf, dstf)
    h1 = _dense(x, agg1p, degp, W1_self, W1_neigh, b1r, relu=True)
    agg2p = _make_sc_agg()(h1, srcf, dstf)
    out = _dense(h1, agg2p, degp, W2_self, W2_neigh, b2r, relu=False)
    return out


# speedup vs baseline: 10.7484x; 1.0281x over previous
import functools

import jax
import jax.numpy as jnp
from jax import lax
from jax.experimental import pallas as pl
from jax.experimental.pallas import tpu as pltpu
from jax.experimental.pallas import tpu_sc as plsc

N = 10000
NP = 10240
E = 320000
D = 128
NC = 2
NS = 16
NW = NC * NS
E_PER_W = E // NW
CH = 80
NCH = E_PER_W // CH
RPS = NP // NS
ZR = 8

_MESH = dict(core_axis_name="c", subcore_axis_name="s",
             num_cores=NC, num_subcores=NS)


def _fill(ref, rows, value):
    @pl.loop(0, rows)
    def _(r):
        @pl.loop(0, D // 16)
        def _(c):
            ref[r, pl.ds(c * 16, 16)] = jnp.full((16,), value, jnp.float32)


def _zero_accum(sid, zrow_v, accum_sh):
    _fill(zrow_v, ZR, 0.0)

    @pl.loop(0, RPS // ZR)
    def _(j):
        pltpu.sync_copy(zrow_v, accum_sh.at[pl.ds(sid * RPS + j * ZR, ZR), :])


def _sc_agg_body(with_deg, h_hbm, srcf_hbm, dstf_hbm, *refs):
    if with_deg:
        (out_hbm, deg_out,
         sidx_v, didx_v, rows0_v, rows1_v, rows2_v, rows3_v,
         zrow_v, accum_sh, *sems) = refs
    else:
        (out_hbm,
         sidx_v, didx_v, rows0_v, rows1_v, rows2_v, rows3_v,
         zrow_v, accum_sh, *sems) = refs
    cid = lax.axis_index("c")
    sid = lax.axis_index("s")
    wid = sid * NC + cid
    ebase = wid * E_PER_W
    rs = pl.ds(sid * RPS, RPS)

    isems = sems[0:8]
    rsems = sems[8:12]
    ssems = sems[12:16]
    rows = (rows0_v, rows1_v, rows2_v, rows3_v)

    _zero_accum(sid, zrow_v, accum_sh)
    if with_deg:
        _fill(rows0_v, CH, 1.0)
    plsc.subcore_barrier()

    def idx_load(m, s):
        off = pl.multiple_of(ebase + m * CH, 8)
        pltpu.async_copy(srcf_hbm.at[pl.ds(off, CH)], sidx_v.at[s], isems[s])
        pltpu.async_copy(dstf_hbm.at[pl.ds(off, CH)], didx_v.at[s], isems[s])

    def idx_wait(s):
        pltpu.make_async_copy(srcf_hbm.at[pl.ds(0, CH)], sidx_v.at[s],
                              isems[s]).wait()
        pltpu.make_async_copy(dstf_hbm.at[pl.ds(0, CH)], didx_v.at[s],
                              isems[s]).wait()

    def gather_start(s, r):
        pltpu.async_copy(h_hbm.at[sidx_v.at[s]], rows[r], rsems[r])

    def gather_wait(r):
        pltpu.make_async_copy(h_hbm.at[sidx_v.at[0]], rows[r], rsems[r]).wait()

    def scatter_start(s, r, buf=None):
        pltpu.async_copy(rows[r if buf is None else buf],
                         accum_sh.at[didx_v.at[s]], ssems[r], add=True)

    def scatter_wait(r):
        pltpu.make_async_copy(rows[0], accum_sh.at[didx_v.at[0]],
                              ssems[r]).wait()

    if with_deg:
        def deg_body(m, guard):
            if m >= 4:
                scatter_wait(m % 4)
                if guard and m + 4 < NCH:
                    idx_load(m + 4, (m + 4) % 8)
            idx_wait(m % 8)
            scatter_start(m % 8, m % 4, buf=0)

        for s in range(8):
            idx_load(s, s)
        for m in range(8):
            deg_body(m, True)

        @pl.loop(1, NCH // 8)
        def _(q):
            for b in range(8):
                def dyn_deg(b=b, q=q):
                    m = q * 8 + b
                    scatter_wait(b % 4)

                    @pl.when(m + 4 < NCH)
                    def _():
                        idx_load(m + 4, (b + 4) % 8)

                    idx_wait(b % 8)
                    scatter_start(b % 8, b % 4, buf=0)

                dyn_deg()

        for m in range((NCH // 8) * 8, NCH):
            deg_body(m, True)
        for k in range(NCH - 4, NCH):
            scatter_wait(k % 4)

        plsc.subcore_barrier()
        pltpu.sync_copy(accum_sh.at[rs, :], deg_out.at[cid, rs, :])
        _zero_accum(sid, zrow_v, accum_sh)
        plsc.subcore_barrier()

    def body(m, guard):
        b8, b4 = m % 8, m % 4
        gather_wait(b4)
        scatter_start(b8, b4)
        if m >= 2:
            scatter_wait((m + 2) % 4)
            if guard and m + 6 < NCH:
                idx_load(m + 6, (m + 6) % 8)
        if guard and m + 2 < NCH:
            idx_wait((m + 2) % 8)
            gather_start((m + 2) % 8, (m + 2) % 4)

    for s in range(8):
        idx_load(s, s)
    idx_wait(0)
    gather_start(0, 0)
    idx_wait(1)
    gather_start(1, 1)

    for m in range(8):
        body(m, True)

    @pl.loop(1, NCH // 8)
    def _(q):
        for b in range(8):
            m = q * 8 + b

            def dyn_body(b=b, m=m):
                b4 = b % 4
                gather_wait(b4)
                scatter_start(b, b4)
                scatter_wait((b + 2) % 4)

                @pl.when(m + 6 < NCH)
                def _():
                    idx_load(m + 6, (b + 6) % 8)

                @pl.when(m + 2 < NCH)
                def _():
                    idx_wait((b + 2) % 8)
                    gather_start((b + 2) % 8, (b + 2) % 4)

            dyn_body()

    for m in range((NCH // 8) * 8, NCH):
        body(m, True)

    scatter_wait((NCH - 2) % 4)
    scatter_wait((NCH - 1) % 4)

    plsc.subcore_barrier()

    pltpu.sync_copy(accum_sh.at[rs, :], out_hbm.at[cid, rs, :])


def _make_sc_agg(with_deg):
    out_type = jax.ShapeDtypeStruct((NC, NP, D), jnp.float32)
    return pl.kernel(
        functools.partial(_sc_agg_body, with_deg),
        out_type=(out_type, out_type) if with_deg else out_type,
        mesh=plsc.VectorSubcoreMesh(**_MESH),
        scratch_types=[
            pltpu.VMEM((8, CH), jnp.int32),
            pltpu.VMEM((8, CH), jnp.int32),
            pltpu.VMEM((CH, D), jnp.float32),
            pltpu.VMEM((CH, D), jnp.float32),
            pltpu.VMEM((CH, D), jnp.float32),
            pltpu.VMEM((CH, D), jnp.float32),
            pltpu.VMEM((ZR, D), jnp.float32),
            pltpu.VMEM_SHARED((NP, D), jnp.float32),
        ] + [pltpu.SemaphoreType.DMA] * 16)


def _dense_body(relu, h_ref, aggp_ref, degp_ref, ws_ref, wn_ref, b_ref, o_ref):
    p = aggp_ref[0] + aggp_ref[1]
    deg = degp_ref[0, :, 0:1] + degp_ref[1, :, 0:1]
    hn = p * (1.0 / jnp.maximum(deg, 1.0))
    acc = jnp.dot(h_ref[...], ws_ref[...], preferred_element_type=jnp.float32)
    acc = acc + jnp.dot(hn, wn_ref[...], preferred_element_type=jnp.float32)
    acc = acc + b_ref[...]
    if relu:
        acc = jnp.maximum(acc, 0.0)
    o_ref[...] = acc


BLK = 1000


def _dense(h, aggp, degp, w_self, w_neigh, b, relu):
    return pl.pallas_call(
        functools.partial(_dense_body, relu),
        grid=(N // BLK,),
        in_specs=[
            pl.BlockSpec((BLK, D), lambda i: (i, 0)),
            pl.BlockSpec((NC, BLK, D), lambda i: (0, i, 0)),
            pl.BlockSpec((NC, BLK, D), lambda i: (0, i, 0)),
            pl.BlockSpec((D, D), lambda i: (0, 0)),
            pl.BlockSpec((D, D), lambda i: (0, 0)),
            pl.BlockSpec((1, D), lambda i: (0, 0)),
        ],
        out_specs=pl.BlockSpec((BLK, D), lambda i: (i, 0)),
        out_shape=jax.ShapeDtypeStruct((N, D), jnp.float32),
    )(h, aggp, degp, w_self, w_neigh, b)


def kernel(x, edge_index, W1_self, W1_neigh, b1, W2_self, W2_neigh, b2):
    srcf = edge_index[0]
    dstf = edge_index[1]
    b1r = b1.reshape(1, D)
    b2r = b2.reshape(1, D)

    agg1p, degp = _make_sc_agg(True)(x, srcf, dstf)
    h1 = _dense(x, agg1p, degp, W1_self, W1_neigh, b1r, relu=True)
    agg2p = _make_sc_agg(False)(h1, srcf, dstf)
    out = _dense(h1, agg2p, degp, W2_self, W2_neigh, b2r, relu=False)
    return out
